# baseline (speedup 1.0000x reference)
import functools
import math

import jax
import jax.numpy as jnp
from jax.experimental import pallas as pl

DIM = 768
HEADS = 12
HDIM = 64
E = 8
TOPK = 2
EXP_DIM = 256
DIM_S = 2048
EPS = 1e-5
THETA = 10000.0
B = 2
S = 2048
N = B * S
L = 2 * S


_QKV_TILE = 512


def _qkv_kernel(x_ref, w_ref, nw_ref, cos_ref, sin_ref, out_ref):
    x = x_ref[...]
    xn = x * jax.lax.rsqrt(jnp.mean(x * x, axis=-1, keepdims=True) + EPS)
    xn = (xn * nw_ref[...]).astype(jnp.bfloat16)
    qkv = jnp.dot(xn, w_ref[...], preferred_element_type=jnp.float32)
    cos = cos_ref[...]
    sin = sin_ref[...]
    q = qkv[:, :DIM] * cos + qkv[:, 3 * DIM:4 * DIM] * sin
    k = qkv[:, DIM:2 * DIM] * cos + qkv[:, 4 * DIM:] * sin
    out_ref[...] = jnp.concatenate(
        [q, k, qkv[:, 2 * DIM:3 * DIM]], axis=-1).astype(jnp.bfloat16)


def _qkv_call(x_flat, w_aug, norm_w, cos_t, sin_t):
    grid = (N // _QKV_TILE,)
    n_pos = S // _QKV_TILE
    return pl.pallas_call(
        _qkv_kernel,
        grid=grid,
        in_specs=[
            pl.BlockSpec((_QKV_TILE, DIM), lambda i: (i, 0)),
            pl.BlockSpec((DIM, 5 * DIM), lambda i: (0, 0)),
            pl.BlockSpec((1, DIM), lambda i: (0, 0)),
            pl.BlockSpec((_QKV_TILE, DIM), lambda i: (i % n_pos, 0)),
            pl.BlockSpec((_QKV_TILE, DIM), lambda i: (i % n_pos, 0)),
        ],
        out_specs=pl.BlockSpec((_QKV_TILE, 3 * DIM), lambda i: (i, 0)),
        out_shape=jax.ShapeDtypeStruct((N, 3 * DIM), jnp.bfloat16),
    )(x_flat, w_aug, norm_w, cos_t, sin_t)


_Q_TILE = 512
_K_TILE = 512


def _attn_kernel(q_ref, k_ref, v_ref, out_ref):
    i = pl.program_id(1)
    c = i // (S // _Q_TILE)
    ib = i % (S // _Q_TILE)
    nb = ib + 1
    scale = 1.0 / math.sqrt(HDIM)
    s_row = (jax.lax.broadcasted_iota(jnp.int32, (_Q_TILE, _K_TILE), 0)
             + ib * _Q_TILE)
    lim0 = s_row
    lim1 = s_row - 1 + c
    cols = jax.lax.broadcasted_iota(jnp.int32, (_Q_TILE, _K_TILE), 1)

    def attend(off):
        q = q_ref[:, off:off + HDIM]

        def body(j, carry):
            m, l, acc = carry
            base = j * _K_TILE
            k0 = k_ref[pl.ds(base, _K_TILE), off:off + HDIM]
            k1 = k_ref[pl.ds(S + base, _K_TILE), off:off + HDIM]
            v0 = v_ref[pl.ds(base, _K_TILE), off:off + HDIM]
            v1 = v_ref[pl.ds(S + base, _K_TILE), off:off + HDIM]
            t = cols + base
            l0 = jax.lax.dot_general(
                q, k0, (((1,), (1,)), ((), ())),
                preferred_element_type=jnp.float32) * scale
            l1 = jax.lax.dot_general(
                q, k1, (((1,), (1,)), ((), ())),
                preferred_element_type=jnp.float32) * scale
            l0, l1 = jax.lax.cond(
                j == nb - 1,
                lambda a, b: (jnp.where(t <= lim0, a, jnp.float32(-1e30)),
                              jnp.where(t <= lim1, b, jnp.float32(-1e30))),
                lambda a, b: (a, b),
                l0, l1)
            m_new = jnp.maximum(m, jnp.maximum(
                jnp.max(l0, axis=-1, keepdims=True),
                jnp.max(l1, axis=-1, keepdims=True)))
            p0 = jnp.exp(l0 - m_new)
            p1 = jnp.exp(l1 - m_new)
            alpha = jnp.exp(m - m_new)
            l_new = l * alpha + jnp.sum(p0, axis=-1, keepdims=True) \
                + jnp.sum(p1, axis=-1, keepdims=True)
            acc_new = acc * alpha \
                + jnp.dot(p0.astype(jnp.bfloat16), v0,
                          preferred_element_type=jnp.float32) \
                + jnp.dot(p1.astype(jnp.bfloat16), v1,
                          preferred_element_type=jnp.float32)
            return m_new, l_new, acc_new

        m0 = jnp.full((_Q_TILE, 1), -1e30, dtype=jnp.float32)
        l0_ = jnp.zeros((_Q_TILE, 1), dtype=jnp.float32)
        a0 = jnp.zeros((_Q_TILE, HDIM), dtype=jnp.float32)
        m, l, acc = jax.lax.fori_loop(0, nb, body, (m0, l0_, a0))
        return acc / l

    out_ref[...] = jnp.concatenate(
        [attend(0), attend(HDIM)], axis=1).astype(jnp.bfloat16)


def _attn_call(qkv):
    grid = (HEADS // 2, N // _Q_TILE)
    return pl.pallas_call(
        _attn_kernel,
        grid=grid,
        in_specs=[
            pl.BlockSpec((_Q_TILE, 2 * HDIM), lambda g, i: (i, g)),
            pl.BlockSpec((N, 2 * HDIM), lambda g, i: (0, HEADS // 2 + g)),
            pl.BlockSpec((N, 2 * HDIM), lambda g, i: (0, HEADS + g)),
        ],
        out_specs=pl.BlockSpec((_Q_TILE, 2 * HDIM), lambda g, i: (i, g)),
        out_shape=jax.ShapeDtypeStruct((N, DIM), jnp.bfloat16),
    )(qkv, qkv, qkv)


_PF_TILE = 512


def _projffn_kernel(o_ref, wo_ref, x_ref, nw_ref, up_ref, down_ref,
                    xffn_ref, base_ref):
    y = jnp.dot(o_ref[...], wo_ref[...], preferred_element_type=jnp.float32)
    resid = y + x_ref[...]
    xn = resid * jax.lax.rsqrt(
        jnp.mean(resid * resid, axis=-1, keepdims=True) + EPS)
    xf = (xn * nw_ref[...]).astype(jnp.bfloat16)
    xffn_ref[...] = xf
    h = jnp.dot(xf, up_ref[...], preferred_element_type=jnp.float32)
    x1 = h[:, :DIM_S]
    x2 = h[:, DIM_S:]
    g = ((x1 * jax.lax.logistic(x1)) * x2).astype(jnp.bfloat16)
    ys = jnp.dot(g, down_ref[...], preferred_element_type=jnp.float32)
    base_ref[...] = ys + resid


def _projffn_call(o_flat, wo_t, x_flat, norm_w, up_t, down_t):
    grid = (N // _PF_TILE,)
    return pl.pallas_call(
        _projffn_kernel,
        grid=grid,
        in_specs=[
            pl.BlockSpec((_PF_TILE, DIM), lambda i: (i, 0)),
            pl.BlockSpec((DIM, DIM), lambda i: (0, 0)),
            pl.BlockSpec((_PF_TILE, DIM), lambda i: (i, 0)),
            pl.BlockSpec((1, DIM), lambda i: (0, 0)),
            pl.BlockSpec((DIM, 2 * DIM_S), lambda i: (0, 0)),
            pl.BlockSpec((DIM_S, DIM), lambda i: (0, 0)),
        ],
        out_specs=[
            pl.BlockSpec((_PF_TILE, DIM), lambda i: (i, 0)),
            pl.BlockSpec((_PF_TILE, DIM), lambda i: (i, 0)),
        ],
        out_shape=[
            jax.ShapeDtypeStruct((N, DIM), jnp.bfloat16),
            jax.ShapeDtypeStruct((N, DIM), jnp.float32),
        ],
    )(o_flat, wo_t, x_flat, norm_w, up_t, down_t)


_MOE_TILE = 512


def _moe_kernel(x_ref, keys_ref, idx_ref, vals_ref, bias_ref,
                w1_ref, w3_ref, w2_ref, base_ref, out_ref):
    x = x_ref[...]
    tok = jnp.dot(x, keys_ref[...], preferred_element_type=jnp.float32)
    idx = idx_ref[0]
    onehot = (idx[:, :, None] ==
              jnp.arange(E, dtype=idx.dtype)[None, None, :]).astype(
                  jnp.float32)
    gathered = jnp.sum(onehot * tok[:, None, :], axis=-1)
    gbias = jnp.sum(onehot * bias_ref[...][None, :, :], axis=-1)
    v = vals_ref[0] + gathered + gbias
    sc = jax.lax.logistic(v)
    sc = sc / jnp.sum(sc, axis=-1, keepdims=True)
    comb = jnp.sum(onehot * sc[:, :, None], axis=1)

    out_ref[...] = base_ref[...]
    for e in range(E):
        h1 = jnp.dot(x, w1_ref[e], preferred_element_type=jnp.float32)
        h3 = jnp.dot(x, w3_ref[e], preferred_element_type=jnp.float32)
        h = ((h1 * jax.lax.logistic(h1)) * h3).astype(jnp.bfloat16)
        y = jax.lax.dot_general(h, w2_ref[e], (((1,), (1,)), ((), ())),
                                preferred_element_type=jnp.float32)
        out_ref[...] += y * comb[:, e:e + 1]


def _moe_call(x_side, keys, idx, vals, bias, experts, base_side):
    grid = (S // _MOE_TILE,)
    nt = S // _MOE_TILE
    w1, w3, w2 = experts[0], experts[1], experts[2]
    idx3 = idx.reshape(nt, _MOE_TILE, TOPK)
    vals3 = vals.reshape(nt, _MOE_TILE, TOPK)
    return pl.pallas_call(
        _moe_kernel,
        grid=grid,
        in_specs=[
            pl.BlockSpec((_MOE_TILE, DIM), lambda i: (i, 0)),
            pl.BlockSpec((DIM, E), lambda i: (0, 0)),
            pl.BlockSpec((1, _MOE_TILE, TOPK), lambda i: (i, 0, 0)),
            pl.BlockSpec((1, _MOE_TILE, TOPK), lambda i: (i, 0, 0)),
            pl.BlockSpec((1, E), lambda i: (0, 0)),
            pl.BlockSpec((E, DIM, EXP_DIM), lambda i: (0, 0, 0)),
            pl.BlockSpec((E, DIM, EXP_DIM), lambda i: (0, 0, 0)),
            pl.BlockSpec((E, DIM, EXP_DIM), lambda i: (0, 0, 0)),
            pl.BlockSpec((_MOE_TILE, DIM), lambda i: (i, 0)),
        ],
        out_specs=pl.BlockSpec((_MOE_TILE, DIM), lambda i: (i, 0)),
        out_shape=jax.ShapeDtypeStruct((S, DIM), jnp.float32),
    )(x_side, keys, idx3, vals3, bias, w1, w3, w2, base_side)


def kernel(x_input, p_indices, p_values, f_indices, f_values, attn_w,
           attn_o_w, attn_norm_w, ffn_norm_w, ffn_up_w, ffn_down_w,
           p_ffn_experts, f_ffn_experts, p_token_keys, f_token_keys,
           p_token_router_bias, f_token_router_bias):
    x_flat = x_input.reshape(N, DIM)

    inv_freq = (1.0 / THETA) ** (
        jnp.arange(0, HDIM, 2, dtype=jnp.float32) / HDIM)
    t = jnp.arange(S, dtype=jnp.float32)
    freqs = jnp.outer(t, inv_freq)
    cos_t = jnp.tile(jnp.concatenate([jnp.cos(freqs)] * 2, axis=1),
                     (1, HEADS))
    sin_t = jnp.tile(jnp.concatenate([jnp.sin(freqs)] * 2, axis=1),
                     (1, HEADS))

    w_t = attn_w.T
    wq, wk, wv = w_t[:, :DIM], w_t[:, DIM:2 * DIM], w_t[:, 2 * DIM:]

    def p_rot(w):
        w4 = w.reshape(DIM, HEADS, 2, HDIM // 2)
        return jnp.stack([w4[:, :, 1], -w4[:, :, 0]],
                         axis=2).reshape(DIM, DIM)

    w_aug = jnp.concatenate([wq, wk, wv, p_rot(wq), p_rot(wk)],
                            axis=1).astype(jnp.bfloat16)

    qkv = _qkv_call(x_flat, w_aug, attn_norm_w.reshape(1, DIM), cos_t, sin_t)

    o_flat = _attn_call(qkv)

    x_ffn, base = _projffn_call(
        o_flat, attn_o_w.T.astype(jnp.bfloat16), x_flat,
        ffn_norm_w.reshape(1, DIM), ffn_up_w.T.astype(jnp.bfloat16),
        ffn_down_w.T.astype(jnp.bfloat16))

    def side(x_side, idx, vals, keys, bias, experts, base_side):
        return _moe_call(x_side, keys.astype(jnp.bfloat16), idx, vals,
                         bias.reshape(1, E), experts.astype(jnp.bfloat16),
                         base_side)

    py = side(x_ffn[:S], p_indices, p_values, p_token_keys,
              p_token_router_bias, p_ffn_experts, base[:S])
    fy = side(x_ffn[S:], f_indices, f_values, f_token_keys,
              f_token_router_bias, f_ffn_experts, base[S:])
    return jnp.concatenate([py, fy], axis=0).reshape(B, S, DIM)


# speedup vs baseline: 1.2344x; 1.2344x over previous
import math

import jax
import jax.numpy as jnp
from jax.experimental import pallas as pl

DIM = 768
HEADS = 12
HDIM = 64
E = 8
TOPK = 2
EXP_DIM = 256
DIM_S = 2048
EPS = 1e-5
THETA = 10000.0
B = 2
S = 2048
N = B * S
L = 2 * S


_QKV_TILE = 256


def _qkv_kernel(x_ref, w_ref, nw_ref, cos_ref, sin_ref, out_ref):
    x = x_ref[...]
    xn = x * jax.lax.rsqrt(jnp.mean(x * x, axis=-1, keepdims=True) + EPS)
    xn = (xn * nw_ref[...]).astype(jnp.bfloat16)
    qkv = jnp.dot(xn, w_ref[...], preferred_element_type=jnp.float32)
    cos = cos_ref[...]
    sin = sin_ref[...]
    q = qkv[:, :DIM] * cos + qkv[:, 3 * DIM:4 * DIM] * sin
    k = qkv[:, DIM:2 * DIM] * cos + qkv[:, 4 * DIM:] * sin
    out_ref[...] = jnp.concatenate(
        [q, k, qkv[:, 2 * DIM:3 * DIM]], axis=-1).astype(jnp.bfloat16)


def _qkv_call(x_flat, w_aug, norm_w, cos_t, sin_t):
    grid = (N // _QKV_TILE,)
    n_pos = S // _QKV_TILE
    return pl.pallas_call(
        _qkv_kernel,
        grid=grid,
        in_specs=[
            pl.BlockSpec((_QKV_TILE, DIM), lambda i: (i, 0)),
            pl.BlockSpec((DIM, 5 * DIM), lambda i: (0, 0)),
            pl.BlockSpec((1, DIM), lambda i: (0, 0)),
            pl.BlockSpec((_QKV_TILE, DIM), lambda i: (i % n_pos, 0)),
            pl.BlockSpec((_QKV_TILE, DIM), lambda i: (i % n_pos, 0)),
        ],
        out_specs=pl.BlockSpec((_QKV_TILE, 3 * DIM), lambda i: (i, 0)),
        out_shape=jax.ShapeDtypeStruct((N, 3 * DIM), jnp.bfloat16),
    )(x_flat, w_aug, norm_w, cos_t, sin_t)


_Q_TILE = 512
_K_TILE = 512


def _attn_kernel(q_ref, k_ref, v_ref, out_ref):
    i = pl.program_id(1)
    c = i // (S // _Q_TILE)
    ib = i % (S // _Q_TILE)
    nb = ib + 1
    scale = 1.0 / math.sqrt(HDIM)
    s_row = (jax.lax.broadcasted_iota(jnp.int32, (_Q_TILE, _K_TILE), 0)
             + ib * _Q_TILE)
    lim0 = s_row
    lim1 = s_row - 1 + c
    cols = jax.lax.broadcasted_iota(jnp.int32, (_Q_TILE, _K_TILE), 1)

    def attend(off):
        q = q_ref[:, off:off + HDIM]

        def body(j, carry):
            m, l, acc = carry
            base = j * _K_TILE
            k0 = k_ref[pl.ds(base, _K_TILE), off:off + HDIM]
            k1 = k_ref[pl.ds(S + base, _K_TILE), off:off + HDIM]
            v0 = v_ref[pl.ds(base, _K_TILE), off:off + HDIM]
            v1 = v_ref[pl.ds(S + base, _K_TILE), off:off + HDIM]
            t = cols + base
            l0 = jax.lax.dot_general(
                q, k0, (((1,), (1,)), ((), ())),
                preferred_element_type=jnp.float32) * scale
            l1 = jax.lax.dot_general(
                q, k1, (((1,), (1,)), ((), ())),
                preferred_element_type=jnp.float32) * scale
            l0 = jnp.where(t <= lim0, l0, jnp.float32(-1e30))
            l1 = jnp.where(t <= lim1, l1, jnp.float32(-1e30))
            m_new = jnp.maximum(m, jnp.maximum(
                jnp.max(l0, axis=-1, keepdims=True),
                jnp.max(l1, axis=-1, keepdims=True)))
            p0 = jnp.exp(l0 - m_new)
            p1 = jnp.exp(l1 - m_new)
            alpha = jnp.exp(m - m_new)
            l_new = l * alpha + jnp.sum(p0, axis=-1, keepdims=True) \
                + jnp.sum(p1, axis=-1, keepdims=True)
            acc_new = acc * alpha \
                + jnp.dot(p0.astype(jnp.bfloat16), v0,
                          preferred_element_type=jnp.float32) \
                + jnp.dot(p1.astype(jnp.bfloat16), v1,
                          preferred_element_type=jnp.float32)
            return m_new, l_new, acc_new

        m0 = jnp.full((_Q_TILE, 1), -1e30, dtype=jnp.float32)
        l0_ = jnp.zeros((_Q_TILE, 1), dtype=jnp.float32)
        a0 = jnp.zeros((_Q_TILE, HDIM), dtype=jnp.float32)
        m, l, acc = jax.lax.fori_loop(0, nb, body, (m0, l0_, a0))
        return acc / l

    out_ref[...] = jnp.concatenate(
        [attend(0), attend(HDIM)], axis=1).astype(jnp.bfloat16)


def _attn_call(qkv):
    grid = (HEADS // 2, N // _Q_TILE)
    return pl.pallas_call(
        _attn_kernel,
        grid=grid,
        in_specs=[
            pl.BlockSpec((_Q_TILE, 2 * HDIM), lambda g, i: (i, g)),
            pl.BlockSpec((N, 2 * HDIM), lambda g, i: (0, HEADS // 2 + g)),
            pl.BlockSpec((N, 2 * HDIM), lambda g, i: (0, HEADS + g)),
        ],
        out_specs=pl.BlockSpec((_Q_TILE, 2 * HDIM), lambda g, i: (i, g)),
        out_shape=jax.ShapeDtypeStruct((N, DIM), jnp.bfloat16),
    )(qkv, qkv, qkv)


_PF_TILE = 512


def _projffn_kernel(o_ref, wo_ref, x_ref, nw_ref, up_ref, down_ref,
                    xffn_ref, base_ref):
    y = jnp.dot(o_ref[...], wo_ref[...], preferred_element_type=jnp.float32)
    resid = y + x_ref[...]
    xn = resid * jax.lax.rsqrt(
        jnp.mean(resid * resid, axis=-1, keepdims=True) + EPS)
    xf = (xn * nw_ref[...]).astype(jnp.bfloat16)
    xffn_ref[...] = xf
    h = jnp.dot(xf, up_ref[...], preferred_element_type=jnp.float32)
    x1 = h[:, :DIM_S]
    x2 = h[:, DIM_S:]
    g = ((x1 * jax.lax.logistic(x1)) * x2).astype(jnp.bfloat16)
    ys = jnp.dot(g, down_ref[...], preferred_element_type=jnp.float32)
    base_ref[...] = ys + resid


def _projffn_call(o_flat, wo_t, x_flat, norm_w, up_t, down_t):
    grid = (N // _PF_TILE,)
    return pl.pallas_call(
        _projffn_kernel,
        grid=grid,
        in_specs=[
            pl.BlockSpec((_PF_TILE, DIM), lambda i: (i, 0)),
            pl.BlockSpec((DIM, DIM), lambda i: (0, 0)),
            pl.BlockSpec((_PF_TILE, DIM), lambda i: (i, 0)),
            pl.BlockSpec((1, DIM), lambda i: (0, 0)),
            pl.BlockSpec((DIM, 2 * DIM_S), lambda i: (0, 0)),
            pl.BlockSpec((DIM_S, DIM), lambda i: (0, 0)),
        ],
        out_specs=[
            pl.BlockSpec((_PF_TILE, DIM), lambda i: (i, 0)),
            pl.BlockSpec((_PF_TILE, DIM), lambda i: (i, 0)),
        ],
        out_shape=[
            jax.ShapeDtypeStruct((N, DIM), jnp.bfloat16),
            jax.ShapeDtypeStruct((N, DIM), jnp.float32),
        ],
    )(o_flat, wo_t, x_flat, norm_w, up_t, down_t)


_MOE_TILE = 512


def _moe_kernel(x_ref, keys_ref, idx_ref, vals_ref, bias_ref,
                w1_ref, w3_ref, w2_ref, base_ref, out_ref):
    x = x_ref[...]
    tok = jnp.dot(x, keys_ref[...], preferred_element_type=jnp.float32)
    idx = idx_ref[0]
    onehot = (idx[:, :, None] ==
              jnp.arange(E, dtype=idx.dtype)[None, None, :]).astype(
                  jnp.float32)
    gathered = jnp.sum(onehot * tok[:, None, :], axis=-1)
    gbias = jnp.sum(onehot * bias_ref[...][None, :, :], axis=-1)
    v = vals_ref[0] + gathered + gbias
    sc = jax.lax.logistic(v)
    sc = sc / jnp.sum(sc, axis=-1, keepdims=True)
    comb = jnp.sum(onehot * sc[:, :, None], axis=1)

    out_ref[...] = base_ref[...]
    for e in range(E):
        h1 = jnp.dot(x, w1_ref[e], preferred_element_type=jnp.float32)
        h3 = jnp.dot(x, w3_ref[e], preferred_element_type=jnp.float32)
        h = ((h1 * jax.lax.logistic(h1)) * h3).astype(jnp.bfloat16)
        y = jax.lax.dot_general(h, w2_ref[e], (((1,), (1,)), ((), ())),
                                preferred_element_type=jnp.float32)
        out_ref[...] += y * comb[:, e:e + 1]


def _moe_call(x_side, keys, idx, vals, bias, experts, base_side):
    grid = (S // _MOE_TILE,)
    nt = S // _MOE_TILE
    w1, w3, w2 = experts[0], experts[1], experts[2]
    idx3 = idx.reshape(nt, _MOE_TILE, TOPK)
    vals3 = vals.reshape(nt, _MOE_TILE, TOPK)
    return pl.pallas_call(
        _moe_kernel,
        grid=grid,
        in_specs=[
            pl.BlockSpec((_MOE_TILE, DIM), lambda i: (i, 0)),
            pl.BlockSpec((DIM, E), lambda i: (0, 0)),
            pl.BlockSpec((1, _MOE_TILE, TOPK), lambda i: (i, 0, 0)),
            pl.BlockSpec((1, _MOE_TILE, TOPK), lambda i: (i, 0, 0)),
            pl.BlockSpec((1, E), lambda i: (0, 0)),
            pl.BlockSpec((E, DIM, EXP_DIM), lambda i: (0, 0, 0)),
            pl.BlockSpec((E, DIM, EXP_DIM), lambda i: (0, 0, 0)),
            pl.BlockSpec((E, DIM, EXP_DIM), lambda i: (0, 0, 0)),
            pl.BlockSpec((_MOE_TILE, DIM), lambda i: (i, 0)),
        ],
        out_specs=pl.BlockSpec((_MOE_TILE, DIM), lambda i: (i, 0)),
        out_shape=jax.ShapeDtypeStruct((S, DIM), jnp.float32),
    )(x_side, keys, idx3, vals3, bias, w1, w3, w2, base_side)


def kernel(x_input, p_indices, p_values, f_indices, f_values, attn_w,
           attn_o_w, attn_norm_w, ffn_norm_w, ffn_up_w, ffn_down_w,
           p_ffn_experts, f_ffn_experts, p_token_keys, f_token_keys,
           p_token_router_bias, f_token_router_bias):
    x_flat = x_input.reshape(N, DIM)

    inv_freq = (1.0 / THETA) ** (
        jnp.arange(0, HDIM, 2, dtype=jnp.float32) / HDIM)
    t = jnp.arange(S, dtype=jnp.float32)
    freqs = jnp.outer(t, inv_freq)
    cos_t = jnp.tile(jnp.concatenate([jnp.cos(freqs)] * 2, axis=1),
                     (1, HEADS))
    sin_t = jnp.tile(jnp.concatenate([jnp.sin(freqs)] * 2, axis=1),
                     (1, HEADS))

    w_t = attn_w.T
    wq, wk, wv = w_t[:, :DIM], w_t[:, DIM:2 * DIM], w_t[:, 2 * DIM:]

    def p_rot(w):
        w4 = w.reshape(DIM, HEADS, 2, HDIM // 2)
        return jnp.stack([w4[:, :, 1], -w4[:, :, 0]],
                         axis=2).reshape(DIM, DIM)

    w_aug = jnp.concatenate([wq, wk, wv, p_rot(wq), p_rot(wk)],
                            axis=1).astype(jnp.bfloat16)

    qkv = _qkv_call(x_flat, w_aug, attn_norm_w.reshape(1, DIM), cos_t, sin_t)

    o_flat = _attn_call(qkv)

    x_ffn, base = _projffn_call(
        o_flat, attn_o_w.T.astype(jnp.bfloat16), x_flat,
        ffn_norm_w.reshape(1, DIM), ffn_up_w.T.astype(jnp.bfloat16),
        ffn_down_w.T.astype(jnp.bfloat16))

    def side(x_side, idx, vals, keys, bias, experts, base_side):
        return _moe_call(x_side, keys.astype(jnp.bfloat16), idx, vals,
                         bias.reshape(1, E), experts.astype(jnp.bfloat16),
                         base_side)

    py = side(x_ffn[:S], p_indices, p_values, p_token_keys,
              p_token_router_bias, p_ffn_experts, base[:S])
    fy = side(x_ffn[S:], f_indices, f_values, f_token_keys,
              f_token_router_bias, f_ffn_experts, base[S:])
    return jnp.concatenate([py, fy], axis=0).reshape(B, S, DIM)
